# baseline (speedup 1.0000x reference)
import functools

import jax
import jax.numpy as jnp
from jax import lax
from jax.experimental import pallas as pl
from jax.experimental.pallas import tpu as pltpu
from jax.experimental.pallas import tpu_sc as plsc

N = 10000
F = 256
E = 160000
KHOPS = 8

NPAD = 10240
EPAD = 163840
HALF = 128
CH = 128
NTILES = 16
ROWS_PER_TILE = NPAD // NTILES
EDGES_PER_TILE_HOP = EPAD // NTILES
EDGES_PER_TILE_DEG = EPAD // 2 // NTILES


@functools.lru_cache(maxsize=1)
def _sc_kernels():
  mesh = plsc.VectorSubcoreMesh(
      core_axis_name="c", subcore_axis_name="s", num_cores=2,
      num_subcores=NTILES)

  nchunk = EDGES_PER_TILE_HOP // CH

  @functools.partial(
      pl.kernel,
      out_type=jax.ShapeDtypeStruct((2 * NPAD, HALF), jnp.float32),
      mesh=mesh,
      scratch_types=[
          pltpu.VMEM((nchunk // 2, CH), jnp.int32),
          pltpu.VMEM((nchunk // 2, CH), jnp.int32),
          pltpu.VMEM((CH, HALF), jnp.float32),
          pltpu.VMEM((CH, HALF), jnp.float32),
          pltpu.VMEM_SHARED((NPAD, HALF), jnp.float32),
          pltpu.SemaphoreType.DMA,
          pltpu.SemaphoreType.DMA,
          pltpu.SemaphoreType.DMA,
          pltpu.SemaphoreType.DMA,
      ],
  )
  def hop_kernel(y_hbm, src_hbm, dst_hbm, acc_out, srcm, dstm, rows0, rows1,
                 acc, gsem0, gsem1, ssem0, ssem1):
    c = lax.axis_index("c")
    s = lax.axis_index("s")
    nbase = s * ROWS_PER_TILE
    rowoff = c * NPAD

    zeros16 = jnp.zeros((16,), jnp.float32)

    @pl.loop(0, CH)
    def _(r):
      for j in range(HALF // 16):
        rows0[r, pl.ds(j * 16, 16)] = zeros16

    for b in range(ROWS_PER_TILE // CH):
      pltpu.sync_copy(rows0, acc.at[pl.ds(nbase + b * CH, CH)])

    plsc.subcore_barrier()

    nb = nchunk // 2
    for h in range(2):
      pltpu.sync_copy(
          src_hbm.at[pl.ds(c * (EPAD // CH) + s * nchunk + h * nb, nb)],
          srcm)
      pltpu.sync_copy(dst_hbm.at[pl.ds(s * nchunk + h * nb, nb)], dstm)

      pltpu.async_copy(y_hbm.at[srcm.at[0]], rows0, gsem0)
      pltpu.async_copy(y_hbm.at[srcm.at[1]], rows1, gsem1)

      @pl.loop(0, nb // 2 - 1)
      def _(k):
        i = 2 * k
        pltpu.make_async_copy(y_hbm.at[srcm.at[i]], rows0, gsem0).wait()
        pltpu.async_copy(rows0, acc.at[dstm.at[i]], ssem0, add=True)
        pltpu.make_async_copy(y_hbm.at[srcm.at[i + 1]], rows1, gsem1).wait()
        pltpu.async_copy(rows1, acc.at[dstm.at[i + 1]], ssem1, add=True)
        pltpu.make_async_copy(rows0, acc.at[dstm.at[i]], ssem0).wait()
        pltpu.async_copy(y_hbm.at[srcm.at[i + 2]], rows0, gsem0)
        pltpu.make_async_copy(rows1, acc.at[dstm.at[i + 1]], ssem1).wait()
        pltpu.async_copy(y_hbm.at[srcm.at[i + 3]], rows1, gsem1)

      pltpu.make_async_copy(y_hbm.at[srcm.at[nb - 2]], rows0, gsem0).wait()
      pltpu.async_copy(rows0, acc.at[dstm.at[nb - 2]], ssem0, add=True)
      pltpu.make_async_copy(y_hbm.at[srcm.at[nb - 1]], rows1, gsem1).wait()
      pltpu.async_copy(rows1, acc.at[dstm.at[nb - 1]], ssem1, add=True)
      pltpu.make_async_copy(rows0, acc.at[dstm.at[nb - 2]], ssem0).wait()
      pltpu.make_async_copy(rows1, acc.at[dstm.at[nb - 1]], ssem1).wait()

    plsc.subcore_barrier()

    pltpu.sync_copy(acc.at[pl.ds(nbase, ROWS_PER_TILE)],
                    acc_out.at[pl.ds(rowoff + nbase, ROWS_PER_TILE)])

  return hop_kernel


_BLK = 256


def _init_body(f_ref, d_ref, dinv_ref, y0_ref, y1_ref):
  dinv = lax.rsqrt(d_ref[...])
  dinv_ref[...] = dinv
  y = f_ref[...] * dinv
  y0_ref[...] = y[:, :HALF]
  y1_ref[...] = y[:, HALF:]


_init_call = pl.pallas_call(
    _init_body,
    grid=(NPAD // _BLK,),
    in_specs=[
        pl.BlockSpec((_BLK, F), lambda i: (i, 0)),
        pl.BlockSpec((_BLK, 1), lambda i: (i, 0)),
    ],
    out_specs=[
        pl.BlockSpec((_BLK, 1), lambda i: (i, 0)),
        pl.BlockSpec((_BLK, HALF), lambda i: (i, 0)),
        pl.BlockSpec((_BLK, HALF), lambda i: (i, 0)),
    ],
    out_shape=[
        jax.ShapeDtypeStruct((NPAD, 1), jnp.float32),
        jax.ShapeDtypeStruct((NPAD, HALF), jnp.float32),
        jax.ShapeDtypeStruct((NPAD, HALF), jnp.float32),
    ],
)


def _finish_body(acc_ref, y_ref, dinv_ref, ynext_ref, x_ref):
  dinv = dinv_ref[...]
  x = dinv * (acc_ref[...] + y_ref[...])
  x_ref[...] = x
  ynext_ref[...] = dinv * x


_finish_call = pl.pallas_call(
    _finish_body,
    grid=(2 * NPAD // _BLK,),
    in_specs=[
        pl.BlockSpec((_BLK, HALF), lambda i: (i, 0)),
        pl.BlockSpec((_BLK, HALF), lambda i: (i, 0)),
        pl.BlockSpec((_BLK, 1), lambda i: (i % (NPAD // _BLK), 0)),
    ],
    out_specs=[
        pl.BlockSpec((_BLK, HALF), lambda i: (i, 0)),
        pl.BlockSpec((_BLK, HALF), lambda i: (i, 0)),
    ],
    out_shape=[
        jax.ShapeDtypeStruct((2 * NPAD, HALF), jnp.float32),
        jax.ShapeDtypeStruct((2 * NPAD, HALF), jnp.float32),
    ],
)


def kernel(feature, edge_index):
  hop_kernel = _sc_kernels()
  src = edge_index[0].astype(jnp.int32)
  dst = edge_index[1].astype(jnp.int32)
  fp = jnp.zeros((NPAD, F), feature.dtype).at[:N].set(feature)
  pad = jnp.full((EPAD - E,), NPAD - 1, jnp.int32)
  srcp = jnp.concatenate([src, pad])
  dstp = jnp.concatenate([dst, pad]).reshape(EPAD // CH, CH)
  src2 = jnp.concatenate([srcp, srcp + NPAD]).reshape(2 * EPAD // CH, CH)

  ones2 = jnp.ones((2 * NPAD, HALF), jnp.float32)
  dacc = hop_kernel(ones2, src2, dstp)
  degtot = (dacc[:NPAD, 0] + 1.0).reshape(NPAD, 1)
  dinv2, yh0, yh1 = _init_call(fp, degtot)
  y2 = jnp.concatenate([yh0, yh1], axis=0)

  layers = [feature]
  for _ in range(KHOPS):
    acc = hop_kernel(y2, src2, dstp)
    y2, x2 = _finish_call(acc, y2, dinv2)
    layers.append(jnp.concatenate([x2[:N], x2[NPAD:NPAD + N]], axis=1))
  return jnp.concatenate(layers, axis=1)


# speedup vs baseline: 1.1370x; 1.1370x over previous
import functools

import jax
import jax.numpy as jnp
from jax import lax
from jax.experimental import pallas as pl
from jax.experimental.pallas import tpu as pltpu
from jax.experimental.pallas import tpu_sc as plsc

N = 10000
F = 256
E = 160000
KHOPS = 8

NPAD = 10240
EPAD = 163840
HALF = 128
CH = 128
NTILES = 16
ROWS_PER_TILE = NPAD // NTILES
EDGES_PER_TILE_HOP = EPAD // NTILES
EDGES_PER_TILE_DEG = EPAD // 2 // NTILES


@functools.lru_cache(maxsize=1)
def _sc_kernels():
  mesh = plsc.VectorSubcoreMesh(
      core_axis_name="c", subcore_axis_name="s", num_cores=2,
      num_subcores=NTILES)

  nchunk = EDGES_PER_TILE_HOP // CH

  @functools.partial(
      pl.kernel,
      out_type=jax.ShapeDtypeStruct((2 * NPAD, HALF), jnp.float32),
      mesh=mesh,
      scratch_types=[
          pltpu.VMEM((nchunk // 2, CH), jnp.int32),
          pltpu.VMEM((nchunk // 2, CH), jnp.int32),
          pltpu.VMEM((CH, HALF), jnp.float32),
          pltpu.VMEM((CH, HALF), jnp.float32),
          pltpu.VMEM_SHARED((NPAD, HALF), jnp.float32),
          pltpu.SemaphoreType.DMA,
          pltpu.SemaphoreType.DMA,
      ],
  )
  def hop_kernel(y_hbm, src_hbm, dst_hbm, acc_out, srcm, dstm, rows0, rows1,
                 acc, sem0, sem1):
    c = lax.axis_index("c")
    s = lax.axis_index("s")
    nbase = s * ROWS_PER_TILE
    rowoff = c * NPAD

    zeros16 = jnp.zeros((16,), jnp.float32)

    @pl.loop(0, CH)
    def _(r):
      for j in range(HALF // 16):
        rows0[r, pl.ds(j * 16, 16)] = zeros16

    for b in range(ROWS_PER_TILE // CH):
      pltpu.sync_copy(rows0, acc.at[pl.ds(nbase + b * CH, CH)])

    plsc.subcore_barrier()

    nb = nchunk // 2
    for h in range(2):
      pltpu.sync_copy(
          src_hbm.at[pl.ds(c * (EPAD // CH) + s * nchunk + h * nb, nb)],
          srcm)
      pltpu.sync_copy(
          dst_hbm.at[pl.ds(c * (EPAD // CH) + s * nchunk + h * nb, nb)],
          dstm)

      pltpu.async_copy(y_hbm.at[srcm.at[0]], rows0, sem0)
      pltpu.async_copy(y_hbm.at[srcm.at[1]], rows1, sem1)

      @pl.loop(0, nb // 2 - 1)
      def _(k):
        i = 2 * k
        pltpu.make_async_copy(y_hbm.at[srcm.at[i]], rows0, sem0).wait()
        pltpu.sync_copy(rows0, acc.at[dstm.at[i]], add=True)
        pltpu.async_copy(y_hbm.at[srcm.at[i + 2]], rows0, sem0)
        pltpu.make_async_copy(y_hbm.at[srcm.at[i + 1]], rows1, sem1).wait()
        pltpu.sync_copy(rows1, acc.at[dstm.at[i + 1]], add=True)
        pltpu.async_copy(y_hbm.at[srcm.at[i + 3]], rows1, sem1)

      pltpu.make_async_copy(y_hbm.at[srcm.at[nb - 2]], rows0, sem0).wait()
      pltpu.sync_copy(rows0, acc.at[dstm.at[nb - 2]], add=True)
      pltpu.make_async_copy(y_hbm.at[srcm.at[nb - 1]], rows1, sem1).wait()
      pltpu.sync_copy(rows1, acc.at[dstm.at[nb - 1]], add=True)

    plsc.subcore_barrier()

    pltpu.sync_copy(acc.at[pl.ds(nbase, ROWS_PER_TILE)],
                    acc_out.at[pl.ds(rowoff + nbase, ROWS_PER_TILE)])

  return hop_kernel


_BLK = 256


def _init_body(f_ref, d_ref, dinv_ref, y0_ref, y1_ref):
  dinv = lax.rsqrt(d_ref[...])
  dinv_ref[...] = dinv
  y = f_ref[...] * dinv
  y0_ref[...] = y[:, :HALF]
  y1_ref[...] = y[:, HALF:]


_init_call = pl.pallas_call(
    _init_body,
    grid=(NPAD // _BLK,),
    in_specs=[
        pl.BlockSpec((_BLK, F), lambda i: (i, 0)),
        pl.BlockSpec((_BLK, 1), lambda i: (i, 0)),
    ],
    out_specs=[
        pl.BlockSpec((_BLK, 1), lambda i: (i, 0)),
        pl.BlockSpec((_BLK, HALF), lambda i: (i, 0)),
        pl.BlockSpec((_BLK, HALF), lambda i: (i, 0)),
    ],
    out_shape=[
        jax.ShapeDtypeStruct((NPAD, 1), jnp.float32),
        jax.ShapeDtypeStruct((NPAD, HALF), jnp.float32),
        jax.ShapeDtypeStruct((NPAD, HALF), jnp.float32),
    ],
)


def _finish_body(acc_ref, y_ref, dinv_ref, ynext_ref, x_ref):
  dinv = dinv_ref[...]
  x = dinv * (acc_ref[...] + y_ref[...])
  x_ref[...] = x
  ynext_ref[...] = dinv * x


_finish_call = pl.pallas_call(
    _finish_body,
    grid=(2 * NPAD // _BLK,),
    in_specs=[
        pl.BlockSpec((_BLK, HALF), lambda i: (i, 0)),
        pl.BlockSpec((_BLK, HALF), lambda i: (i, 0)),
        pl.BlockSpec((_BLK, 1), lambda i: (i % (NPAD // _BLK), 0)),
    ],
    out_specs=[
        pl.BlockSpec((_BLK, HALF), lambda i: (i, 0)),
        pl.BlockSpec((_BLK, HALF), lambda i: (i, 0)),
    ],
    out_shape=[
        jax.ShapeDtypeStruct((2 * NPAD, HALF), jnp.float32),
        jax.ShapeDtypeStruct((2 * NPAD, HALF), jnp.float32),
    ],
)


def kernel(feature, edge_index):
  hop_kernel = _sc_kernels()
  src = edge_index[0].astype(jnp.int32)
  dst = edge_index[1].astype(jnp.int32)
  fp = jnp.zeros((NPAD, F), feature.dtype).at[:N].set(feature)
  pad = jnp.full((EPAD - E,), NPAD - 1, jnp.int32)
  srcp = jnp.concatenate([src, pad])
  src2 = jnp.concatenate([srcp, srcp + NPAD]).reshape(2 * EPAD // CH, CH)
  dstp = jnp.concatenate([dst, pad])
  dst2 = jnp.concatenate([dstp, dstp]).reshape(2 * EPAD // CH, CH)

  e2 = E // 2
  padh = jnp.full((EPAD - e2,), NPAD - 1, jnp.int32)
  dst_deg = jnp.concatenate([dst[:e2], padh, dst[e2:], padh])
  dst_deg = dst_deg.reshape(2 * EPAD // CH, CH)
  ones2 = jnp.ones((2 * NPAD, HALF), jnp.float32)
  dacc = hop_kernel(ones2, src2, dst_deg)
  degtot = (dacc[:NPAD, 0] + dacc[NPAD:, 0] + 1.0).reshape(NPAD, 1)
  dinv2, yh0, yh1 = _init_call(fp, degtot)
  y2 = jnp.concatenate([yh0, yh1], axis=0)

  layers = [feature]
  for _ in range(KHOPS):
    acc = hop_kernel(y2, src2, dst2)
    y2, x2 = _finish_call(acc, y2, dinv2)
    layers.append(jnp.concatenate([x2[:N], x2[NPAD:NPAD + N]], axis=1))
  return jnp.concatenate(layers, axis=1)
